# Initial kernel scaffold; baseline (speedup 1.0000x reference)
import jax
import jax.numpy as jnp
from jax.experimental import pallas as pl


def kernel(user_x, session_x, message_x, feedback_x, session_to_user, message_to_user, feedback_to_user, Wu, bu, Ws, bs, Wm, bm, Wf, bf, Wfu1, bfu1, Wfu2, bfu2, Wc, bc):
    raise NotImplementedError("write your pallas kernel here")


# speedup vs baseline: 1.8132x; 1.8132x over previous
import functools

import jax
import jax.numpy as jnp
from jax import lax
from jax.experimental import pallas as pl
from jax.experimental.pallas import tpu as pltpu
from jax.experimental.pallas import tpu_sc as plsc

N_USERS = 50000
N_PAD = 50048
N_EDGE = 500000
D_IN = 128
HID = 64
HALF = 32
QHALF = 16
SUB = 80
KSUB = 10
CHUNK = SUB * KSUB
NCHUNK = N_EDGE // CHUNK
NTILE = 16
ROWS_PER_TILE = N_PAD // NTILE


def _proj(x, W, b):
    N = x.shape[0]
    BLK = 2000

    def body(xr, wr, br, outr):
        acc = lax.dot_general(xr[...], wr[...], (((1,), (1,)), ((), ())),
                              preferred_element_type=jnp.float32)
        outr[...] = jnp.maximum(acc + br[...], 0.0)

    return pl.pallas_call(
        body,
        grid=(N // BLK,),
        in_specs=[
            pl.BlockSpec((BLK, D_IN), lambda i: (i, 0)),
            pl.BlockSpec((HID, D_IN), lambda i: (0, 0)),
            pl.BlockSpec((1, HID), lambda i: (0, 0)),
        ],
        out_specs=pl.BlockSpec((BLK, HID), lambda i: (i, 0)),
        out_shape=jax.ShapeDtypeStruct((N, HID), jnp.float32),
    )(x, W, b.reshape(1, HID))


def _sc_aggregate(h_s, h_m, h_f, ix_s, ix_m, ix_f):
    mesh = plsc.VectorSubcoreMesh(core_axis_name="c", subcore_axis_name="s")

    @functools.partial(
        pl.kernel,
        mesh=mesh,
        out_type=[
            jax.ShapeDtypeStruct((3, 4, N_PAD, QHALF), jnp.float32),
            jax.ShapeDtypeStruct((N_PAD,), jnp.float32),
            jax.ShapeDtypeStruct((N_PAD,), jnp.float32),
            jax.ShapeDtypeStruct((N_PAD,), jnp.float32),
        ],
        scratch_types=[
            pltpu.VMEM((KSUB, SUB), jnp.int32),
            pltpu.VMEM((CHUNK, QHALF), jnp.float32),
            pltpu.VMEM((CHUNK, QHALF), jnp.float32),
            pltpu.VMEM((CHUNK,), jnp.float32),
            pltpu.VMEM((ROWS_PER_TILE,), jnp.float32),
            pltpu.VMEM((SUB,), jnp.float32),
            pltpu.VMEM_SHARED((N_PAD, QHALF), jnp.float32),
            pltpu.VMEM_SHARED((N_PAD,), jnp.float32),
        ],
        compiler_params=pltpu.CompilerParams(use_tc_tiling_on_sc=False),
    )
    def agg(hs, hm, hf, ixs, ixm, ixf, sums, cnt0, cnt1, cnt2,
            idx_v, rows_v, zbuf, z1d, cbuf, ones_v, acc, cnt):
        c = lax.axis_index("c")
        s = lax.axis_index("s")
        zero16 = jnp.zeros((16,), jnp.float32)
        one16 = jnp.ones((16,), jnp.float32)

        def zrow(r, _):
            zbuf[r, pl.ds(0, 16)] = zero16
            return 0
        lax.fori_loop(0, CHUNK, zrow, 0)

        def z1row(q, _):
            z1d[pl.ds(q * 16, 16)] = zero16
            return 0
        lax.fori_loop(0, CHUNK // 16, z1row, 0)

        for q in range(SUB // 16):
            ones_v[pl.ds(q * 16, 16)] = one16

        base = s * ROWS_PER_TILE

        def one_group(t, g, h, ix, cnt_out):
            pltpu.sync_copy(zbuf, acc.at[pl.ds(base, CHUNK)])
            pltpu.sync_copy(zbuf, acc.at[pl.ds(base + CHUNK, CHUNK)])
            pltpu.sync_copy(zbuf, acc.at[pl.ds(base + 2 * CHUNK, CHUNK)])
            pltpu.sync_copy(zbuf.at[pl.ds(0, ROWS_PER_TILE - 3 * CHUNK)],
                            acc.at[pl.ds(base + 3 * CHUNK,
                                         ROWS_PER_TILE - 3 * CHUNK)])

            if g == 0:
                @pl.when(c == t % 2)
                def _():
                    for q in range(3):
                        pltpu.sync_copy(
                            z1d, cnt.at[pl.ds(base + q * CHUNK, CHUNK)])
                    pltpu.sync_copy(
                        z1d.at[pl.ds(0, ROWS_PER_TILE - 3 * CHUNK)],
                        cnt.at[pl.ds(base + 3 * CHUNK,
                                     ROWS_PER_TILE - 3 * CHUNK)])

            plsc.subcore_barrier()

            nloc = (NCHUNK - s + NTILE - 1) // NTILE
            col0 = c * HALF + g * QHALF

            def chunk_body(i, _):
                j = s + i * NTILE
                pltpu.sync_copy(ix.at[j], idx_v)
                pltpu.sync_copy(
                    h.at[pl.ds(j * CHUNK, CHUNK), pl.ds(col0, QHALF)],
                    rows_v)
                for k in range(KSUB):
                    pltpu.sync_copy(rows_v.at[pl.ds(k * SUB, SUB)],
                                    acc.at[idx_v.at[k]], add=True)

                if g == 0:
                    @pl.when(c == t % 2)
                    def _():
                        for k in range(KSUB):
                            pltpu.sync_copy(ones_v, cnt.at[idx_v.at[k]],
                                            add=True)
                return 0

            lax.fori_loop(0, nloc, chunk_body, 0)
            plsc.subcore_barrier()

            quarter = 2 * c + g
            pltpu.sync_copy(
                acc.at[pl.ds(base, ROWS_PER_TILE)],
                sums.at[t, quarter, pl.ds(base, ROWS_PER_TILE)])

            if g == 0:
                @pl.when(c == t % 2)
                def _():
                    pltpu.sync_copy(cnt.at[pl.ds(base, ROWS_PER_TILE)],
                                    cbuf)
                    pltpu.sync_copy(
                        cbuf, cnt_out.at[pl.ds(base, ROWS_PER_TILE)])

            plsc.subcore_barrier()

        for t, (h, ix, cnt_out) in enumerate(
                [(hs, ixs, cnt0), (hm, ixm, cnt1), (hf, ixf, cnt2)]):
            for g in range(2):
                one_group(t, g, h, ix, cnt_out)

    return agg(h_s, h_m, h_f, ix_s, ix_m, ix_f)


def _mlp(user_x, sums, cnts3, Wu, bu, Wfu1, bfu1, Wfu2, bfu2, Wc, bc):
    BLK = 1000

    def body(uxr, sr, cr, wur, bur, w1r, b1r, w2r, b2r, wcr, bcr,
             logitr, embr):
        uh = jnp.maximum(
            lax.dot_general(uxr[...], wur[...], (((1,), (1,)), ((), ())),
                            preferred_element_type=jnp.float32) + bur[...],
            0.0)
        means = []
        for t in range(3):
            m = jnp.concatenate([sr[t, 0], sr[t, 1], sr[t, 2], sr[t, 3]],
                                axis=1)
            cl = jnp.maximum(cr[t], 1.0)
            means.append(m / cl)
        fused = jnp.concatenate([uh] + means, axis=1)
        h = jnp.maximum(
            lax.dot_general(fused, w1r[...], (((1,), (1,)), ((), ())),
                            preferred_element_type=jnp.float32) + b1r[...],
            0.0)
        e = jnp.maximum(
            lax.dot_general(h, w2r[...], (((1,), (1,)), ((), ())),
                            preferred_element_type=jnp.float32) + b2r[...],
            0.0)
        embr[...] = e
        logitr[...] = jnp.sum(e * wcr[...], axis=1, keepdims=True) + bcr[...]

    logits2d, emb = pl.pallas_call(
        body,
        grid=(N_USERS // BLK,),
        in_specs=[
            pl.BlockSpec((BLK, D_IN), lambda i: (i, 0)),
            pl.BlockSpec((3, 4, BLK, QHALF), lambda i: (0, 0, i, 0)),
            pl.BlockSpec((3, BLK, 1), lambda i: (0, i, 0)),
            pl.BlockSpec((HID, D_IN), lambda i: (0, 0)),
            pl.BlockSpec((1, HID), lambda i: (0, 0)),
            pl.BlockSpec((HID, 4 * HID), lambda i: (0, 0)),
            pl.BlockSpec((1, HID), lambda i: (0, 0)),
            pl.BlockSpec((HID, HID), lambda i: (0, 0)),
            pl.BlockSpec((1, HID), lambda i: (0, 0)),
            pl.BlockSpec((1, HID), lambda i: (0, 0)),
            pl.BlockSpec((1, 1), lambda i: (0, 0)),
        ],
        out_specs=[
            pl.BlockSpec((BLK, 1), lambda i: (i, 0)),
            pl.BlockSpec((BLK, HID), lambda i: (i, 0)),
        ],
        out_shape=[
            jax.ShapeDtypeStruct((N_USERS, 1), jnp.float32),
            jax.ShapeDtypeStruct((N_USERS, HID), jnp.float32),
        ],
    )(user_x, sums, cnts3, Wu, bu.reshape(1, HID),
      Wfu1, bfu1.reshape(1, HID), Wfu2, bfu2.reshape(1, HID), Wc,
      bc.reshape(1, 1))
    return logits2d.reshape(N_USERS), emb


def kernel(user_x, session_x, message_x, feedback_x, session_to_user,
           message_to_user, feedback_to_user, Wu, bu, Ws, bs, Wm, bm, Wf, bf,
           Wfu1, bfu1, Wfu2, bfu2, Wc, bc):
    h_s = _proj(session_x, Ws, bs)
    h_m = _proj(message_x, Wm, bm)
    h_f = _proj(feedback_x, Wf, bf)
    ix_s = session_to_user.reshape(NCHUNK, KSUB, SUB)
    ix_m = message_to_user.reshape(NCHUNK, KSUB, SUB)
    ix_f = feedback_to_user.reshape(NCHUNK, KSUB, SUB)
    sums, c0, c1, c2 = _sc_aggregate(h_s, h_m, h_f, ix_s, ix_m, ix_f)
    cnts3 = jnp.stack([c0, c1, c2]).reshape(3, N_PAD, 1)
    return _mlp(user_x, sums, cnts3, Wu, bu, Wfu1, bfu1, Wfu2, bfu2, Wc, bc)


# speedup vs baseline: 2.8492x; 1.5714x over previous
import functools

import jax
import jax.numpy as jnp
from jax import lax
from jax.experimental import pallas as pl
from jax.experimental.pallas import tpu as pltpu
from jax.experimental.pallas import tpu_sc as plsc

N_USERS = 50000
N_PAD = 50048
N_EDGE = 500000
N_EDGE2 = N_EDGE // 2
D_IN = 128
HID = 64
HALF = 32
QHALF = 16
SUB = 125
KSUB = 8
CHUNK = SUB * KSUB
NCHUNK = N_EDGE // CHUNK
NCHUNK2 = NCHUNK // 2
NTILE = 16
ROWS_PER_TILE = N_PAD // NTILE


def _proj(x, W, b):
    N = x.shape[0]
    N2 = N // 2
    BLK = 2000
    NBLK2 = N2 // BLK

    def body(x1r, x2r, wr, br, outr):
        a1 = lax.dot_general(x1r[...], wr[...], (((1,), (1,)), ((), ())),
                             preferred_element_type=jnp.float32)
        outr[:, 0:HID] = jnp.maximum(a1 + br[...], 0.0)
        a2 = lax.dot_general(x2r[...], wr[...], (((1,), (1,)), ((), ())),
                             preferred_element_type=jnp.float32)
        outr[:, HID:2 * HID] = jnp.maximum(a2 + br[...], 0.0)

    return pl.pallas_call(
        body,
        grid=(NBLK2,),
        in_specs=[
            pl.BlockSpec((BLK, D_IN), lambda i: (i, 0)),
            pl.BlockSpec((BLK, D_IN), lambda i: (NBLK2 + i, 0)),
            pl.BlockSpec((HID, D_IN), lambda i: (0, 0)),
            pl.BlockSpec((1, HID), lambda i: (0, 0)),
        ],
        out_specs=pl.BlockSpec((BLK, 2 * HID), lambda i: (i, 0)),
        out_shape=jax.ShapeDtypeStruct((N2, 2 * HID), jnp.float32),
    )(x, x, W, b.reshape(1, HID))


def _sc_aggregate(h_s, h_m, h_f, ix_s, ix_m, ix_f):
    mesh = plsc.VectorSubcoreMesh(core_axis_name="c", subcore_axis_name="s")

    @functools.partial(
        pl.kernel,
        mesh=mesh,
        out_type=[
            jax.ShapeDtypeStruct((3, 4, N_PAD, QHALF), jnp.float32),
            jax.ShapeDtypeStruct((N_PAD,), jnp.float32),
            jax.ShapeDtypeStruct((N_PAD,), jnp.float32),
            jax.ShapeDtypeStruct((N_PAD,), jnp.float32),
        ],
        scratch_types=[
            pltpu.VMEM((KSUB, SUB), jnp.int32),
            pltpu.VMEM((CHUNK, QHALF), jnp.float32),
            pltpu.VMEM((CHUNK, QHALF), jnp.float32),
            pltpu.VMEM((1024,), jnp.float32),
            pltpu.VMEM((ROWS_PER_TILE,), jnp.float32),
            pltpu.VMEM((128,), jnp.float32),
            pltpu.VMEM_SHARED((N_PAD, QHALF), jnp.float32),
            pltpu.VMEM_SHARED((N_PAD,), jnp.float32),
        ],
        compiler_params=pltpu.CompilerParams(use_tc_tiling_on_sc=False),
    )
    def agg(hs, hm, hf, ixs, ixm, ixf, sums, cnt0, cnt1, cnt2,
            idx_v, rows_v, zbuf, z1d, cbuf, ones_v, acc, cnt):
        c = lax.axis_index("c")
        s = lax.axis_index("s")
        zero16 = jnp.zeros((16,), jnp.float32)
        one16 = jnp.ones((16,), jnp.float32)

        def zrow(r, _):
            zbuf[r, pl.ds(0, 16)] = zero16
            return 0
        lax.fori_loop(0, CHUNK, zrow, 0)

        def z1row(q, _):
            z1d[pl.ds(q * 16, 16)] = zero16
            return 0
        lax.fori_loop(0, 1024 // 16, z1row, 0)

        for q in range(128 // 16):
            ones_v[pl.ds(q * 16, 16)] = one16

        base = s * ROWS_PER_TILE

        def one_group(t, g, h, ix, cnt_out):
            pltpu.sync_copy(zbuf, acc.at[pl.ds(base, CHUNK)])
            pltpu.sync_copy(zbuf, acc.at[pl.ds(base + CHUNK, CHUNK)])
            pltpu.sync_copy(zbuf, acc.at[pl.ds(base + 2 * CHUNK, CHUNK)])
            pltpu.sync_copy(zbuf.at[pl.ds(0, ROWS_PER_TILE - 3 * CHUNK)],
                            acc.at[pl.ds(base + 3 * CHUNK,
                                         ROWS_PER_TILE - 3 * CHUNK)])

            if g == 0:
                @pl.when(c == t % 2)
                def _():
                    for q in range(3):
                        pltpu.sync_copy(
                            z1d, cnt.at[pl.ds(base + q * 1024, 1024)])
                    pltpu.sync_copy(
                        z1d.at[pl.ds(0, ROWS_PER_TILE - 3 * 1024)],
                        cnt.at[pl.ds(base + 3 * 1024,
                                     ROWS_PER_TILE - 3 * 1024)])

            plsc.subcore_barrier()

            nloc = (NCHUNK - s + NTILE - 1) // NTILE
            col0 = c * HALF + g * QHALF

            def chunk_body(i, _):
                j = s + i * NTILE
                pltpu.sync_copy(ix.at[j], idx_v)
                row = jnp.where(j < NCHUNK2, j, j - NCHUNK2) * CHUNK
                coff = jnp.where(j < NCHUNK2, col0, HID + col0)
                pltpu.sync_copy(
                    h.at[pl.ds(row, CHUNK), pl.ds(coff, QHALF)],
                    rows_v)
                for k in range(KSUB):
                    pltpu.sync_copy(rows_v.at[pl.ds(k * SUB, SUB)],
                                    acc.at[idx_v.at[k]], add=True)

                if g == 0:
                    @pl.when(c == t % 2)
                    def _():
                        for k in range(KSUB):
                            pltpu.sync_copy(ones_v.at[pl.ds(0, SUB)],
                                            cnt.at[idx_v.at[k]], add=True)
                return 0

            lax.fori_loop(0, nloc, chunk_body, 0)
            plsc.subcore_barrier()

            quarter = 2 * c + g
            pltpu.sync_copy(
                acc.at[pl.ds(base, ROWS_PER_TILE)],
                sums.at[t, quarter, pl.ds(base, ROWS_PER_TILE)])

            if g == 0:
                @pl.when(c == t % 2)
                def _():
                    pltpu.sync_copy(cnt.at[pl.ds(base, ROWS_PER_TILE)],
                                    cbuf)
                    pltpu.sync_copy(
                        cbuf, cnt_out.at[pl.ds(base, ROWS_PER_TILE)])

            plsc.subcore_barrier()

        for t, (h, ix, cnt_out) in enumerate(
                [(hs, ixs, cnt0), (hm, ixm, cnt1), (hf, ixf, cnt2)]):
            for g in range(2):
                one_group(t, g, h, ix, cnt_out)

    return agg(h_s, h_m, h_f, ix_s, ix_m, ix_f)


def _mlp(user_x, sums, cnts3, Wu, bu, Wfu1, bfu1, Wfu2, bfu2, Wc, bc):
    BLK = 1000

    def body(uxr, sr, cr, wur, bur, w1r, b1r, w2r, b2r, wcr, bcr,
             logitr, embr):
        uh = jnp.maximum(
            lax.dot_general(uxr[...], wur[...], (((1,), (1,)), ((), ())),
                            preferred_element_type=jnp.float32) + bur[...],
            0.0)
        means = []
        for t in range(3):
            m = jnp.concatenate([sr[t, 0], sr[t, 1], sr[t, 2], sr[t, 3]],
                                axis=1)
            cl = jnp.maximum(cr[t], 1.0)
            means.append(m / cl)
        fused = jnp.concatenate([uh] + means, axis=1)
        h = jnp.maximum(
            lax.dot_general(fused, w1r[...], (((1,), (1,)), ((), ())),
                            preferred_element_type=jnp.float32) + b1r[...],
            0.0)
        e = jnp.maximum(
            lax.dot_general(h, w2r[...], (((1,), (1,)), ((), ())),
                            preferred_element_type=jnp.float32) + b2r[...],
            0.0)
        embr[...] = e
        logitr[...] = jnp.sum(e * wcr[...], axis=1, keepdims=True) + bcr[...]

    logits2d, emb = pl.pallas_call(
        body,
        grid=(N_USERS // BLK,),
        in_specs=[
            pl.BlockSpec((BLK, D_IN), lambda i: (i, 0)),
            pl.BlockSpec((3, 4, BLK, QHALF), lambda i: (0, 0, i, 0)),
            pl.BlockSpec((3, BLK, 1), lambda i: (0, i, 0)),
            pl.BlockSpec((HID, D_IN), lambda i: (0, 0)),
            pl.BlockSpec((1, HID), lambda i: (0, 0)),
            pl.BlockSpec((HID, 4 * HID), lambda i: (0, 0)),
            pl.BlockSpec((1, HID), lambda i: (0, 0)),
            pl.BlockSpec((HID, HID), lambda i: (0, 0)),
            pl.BlockSpec((1, HID), lambda i: (0, 0)),
            pl.BlockSpec((1, HID), lambda i: (0, 0)),
            pl.BlockSpec((1, 1), lambda i: (0, 0)),
        ],
        out_specs=[
            pl.BlockSpec((BLK, 1), lambda i: (i, 0)),
            pl.BlockSpec((BLK, HID), lambda i: (i, 0)),
        ],
        out_shape=[
            jax.ShapeDtypeStruct((N_USERS, 1), jnp.float32),
            jax.ShapeDtypeStruct((N_USERS, HID), jnp.float32),
        ],
    )(user_x, sums, cnts3, Wu, bu.reshape(1, HID),
      Wfu1, bfu1.reshape(1, HID), Wfu2, bfu2.reshape(1, HID), Wc,
      bc.reshape(1, 1))
    return logits2d.reshape(N_USERS), emb


def kernel(user_x, session_x, message_x, feedback_x, session_to_user,
           message_to_user, feedback_to_user, Wu, bu, Ws, bs, Wm, bm, Wf, bf,
           Wfu1, bfu1, Wfu2, bfu2, Wc, bc):
    h_s = _proj(session_x, Ws, bs)
    h_m = _proj(message_x, Wm, bm)
    h_f = _proj(feedback_x, Wf, bf)
    ix_s = session_to_user.reshape(NCHUNK, KSUB, SUB)
    ix_m = message_to_user.reshape(NCHUNK, KSUB, SUB)
    ix_f = feedback_to_user.reshape(NCHUNK, KSUB, SUB)
    sums, c0, c1, c2 = _sc_aggregate(h_s, h_m, h_f, ix_s, ix_m, ix_f)
    cnts3 = jnp.stack([c0, c1, c2]).reshape(3, N_PAD, 1)
    return _mlp(user_x, sums, cnts3, Wu, bu, Wfu1, bfu1, Wfu2, bfu2, Wc, bc)


# speedup vs baseline: 3.6810x; 1.2920x over previous
import functools

import jax
import jax.numpy as jnp
from jax import lax
from jax.experimental import pallas as pl
from jax.experimental.pallas import tpu as pltpu
from jax.experimental.pallas import tpu_sc as plsc

N_USERS = 50000
N_PAD = 50048
N_EDGE = 500000
N_EDGE2 = N_EDGE // 2
D_IN = 128
HID = 64
HALF = 32
QHALF = 16
SUB = 125
KSUB = 8
CHUNK = SUB * KSUB
NCHUNK = N_EDGE // CHUNK
NCHUNK2 = NCHUNK // 2
NTILE = 16
ROWS_PER_TILE = N_PAD // NTILE


def _proj(x, W, b):
    N = x.shape[0]
    N2 = N // 2
    BLK = 2000
    NBLK2 = N2 // BLK

    def body(x1r, x2r, wr, br, outr):
        a1 = lax.dot_general(x1r[...], wr[...], (((1,), (1,)), ((), ())),
                             preferred_element_type=jnp.float32)
        outr[:, 0:HID] = jnp.maximum(a1 + br[...], 0.0)
        a2 = lax.dot_general(x2r[...], wr[...], (((1,), (1,)), ((), ())),
                             preferred_element_type=jnp.float32)
        outr[:, HID:2 * HID] = jnp.maximum(a2 + br[...], 0.0)

    return pl.pallas_call(
        body,
        grid=(NBLK2,),
        in_specs=[
            pl.BlockSpec((BLK, D_IN), lambda i: (i, 0)),
            pl.BlockSpec((BLK, D_IN), lambda i: (NBLK2 + i, 0)),
            pl.BlockSpec((HID, D_IN), lambda i: (0, 0)),
            pl.BlockSpec((1, HID), lambda i: (0, 0)),
        ],
        out_specs=pl.BlockSpec((BLK, 2 * HID), lambda i: (i, 0)),
        out_shape=jax.ShapeDtypeStruct((N2, 2 * HID), jnp.float32),
    )(x, x, W, b.reshape(1, HID))


def _sc_aggregate(h_s, h_m, h_f, ix_s, ix_m, ix_f):
    mesh = plsc.VectorSubcoreMesh(core_axis_name="c", subcore_axis_name="s")

    @functools.partial(
        pl.kernel,
        mesh=mesh,
        out_type=[
            jax.ShapeDtypeStruct((3, 4, N_PAD, QHALF), jnp.float32),
            jax.ShapeDtypeStruct((N_PAD,), jnp.float32),
            jax.ShapeDtypeStruct((N_PAD,), jnp.float32),
            jax.ShapeDtypeStruct((N_PAD,), jnp.float32),
        ],
        scratch_types=[
            pltpu.VMEM((2, KSUB, SUB), jnp.int32),
            pltpu.VMEM((2, CHUNK, QHALF), jnp.float32),
            pltpu.VMEM((512, QHALF), jnp.float32),
            pltpu.VMEM((512,), jnp.float32),
            pltpu.VMEM((ROWS_PER_TILE,), jnp.float32),
            pltpu.VMEM((128,), jnp.float32),
            pltpu.VMEM_SHARED((N_PAD, QHALF), jnp.float32),
            pltpu.VMEM_SHARED((N_PAD,), jnp.float32),
            pltpu.SemaphoreType.DMA,
            pltpu.SemaphoreType.DMA,
        ],
        compiler_params=pltpu.CompilerParams(use_tc_tiling_on_sc=False),
    )
    def agg(hs, hm, hf, ixs, ixm, ixf, sums, cnt0, cnt1, cnt2,
            idx_v, rows_v, zbuf, z1d, cbuf, ones_v, acc, cnt, sem0, sem1):
        c = lax.axis_index("c")
        s = lax.axis_index("s")
        zero16 = jnp.zeros((16,), jnp.float32)
        one16 = jnp.ones((16,), jnp.float32)

        def zrow(r, _):
            zbuf[r, pl.ds(0, 16)] = zero16
            return 0
        lax.fori_loop(0, 512, zrow, 0)

        def z1row(q, _):
            z1d[pl.ds(q * 16, 16)] = zero16
            return 0
        lax.fori_loop(0, 512 // 16, z1row, 0)

        for q in range(128 // 16):
            ones_v[pl.ds(q * 16, 16)] = one16

        base = s * ROWS_PER_TILE

        def one_group(t, g, h, ix, cnt_out):
            for q in range(6):
                pltpu.sync_copy(zbuf, acc.at[pl.ds(base + q * 512, 512)])
            pltpu.sync_copy(zbuf.at[pl.ds(0, ROWS_PER_TILE - 6 * 512)],
                            acc.at[pl.ds(base + 6 * 512,
                                         ROWS_PER_TILE - 6 * 512)])

            if g == 0:
                @pl.when(c == t % 2)
                def _():
                    for q in range(6):
                        pltpu.sync_copy(
                            z1d, cnt.at[pl.ds(base + q * 512, 512)])
                    pltpu.sync_copy(
                        z1d.at[pl.ds(0, ROWS_PER_TILE - 6 * 512)],
                        cnt.at[pl.ds(base + 6 * 512,
                                     ROWS_PER_TILE - 6 * 512)])

            plsc.subcore_barrier()

            nloc = (NCHUNK - s + NTILE - 1) // NTILE
            col0 = c * HALF + g * QHALF
            sems = (sem0, sem1)

            def start_in(i, slot):
                j = s + i * NTILE
                pltpu.async_copy(ix.at[j], idx_v.at[slot], sems[slot])
                row = jnp.where(j < NCHUNK2, j, j - NCHUNK2) * CHUNK
                coff = jnp.where(j < NCHUNK2, col0, HID + col0)
                pltpu.async_copy(
                    h.at[pl.ds(row, CHUNK), pl.ds(coff, QHALF)],
                    rows_v.at[slot], sems[slot])

            def wait_in(slot):
                pltpu.make_async_copy(
                    ix.at[0], idx_v.at[slot], sems[slot]).wait()
                pltpu.make_async_copy(
                    h.at[pl.ds(0, CHUNK), pl.ds(0, QHALF)],
                    rows_v.at[slot], sems[slot]).wait()

            def scatters(i, slot, t, g):
                for k in range(KSUB):
                    pltpu.sync_copy(rows_v.at[slot, pl.ds(k * SUB, SUB)],
                                    acc.at[idx_v.at[slot, k]], add=True)
                if g == 0:
                    @pl.when(c == t % 2)
                    def _():
                        for k in range(KSUB):
                            pltpu.sync_copy(
                                ones_v.at[pl.ds(0, SUB)],
                                cnt.at[idx_v.at[slot, k]], add=True)

            @pl.when(nloc > 0)
            def _():
                start_in(0, 0)

            def pair_body(p, _):
                i0 = 2 * p

                @pl.when(i0 + 1 < nloc)
                def _():
                    start_in(i0 + 1, 1)
                wait_in(0)
                scatters(i0, 0, t, g)

                @pl.when(i0 + 2 < nloc)
                def _():
                    start_in(i0 + 2, 0)

                @pl.when(i0 + 1 < nloc)
                def _():
                    wait_in(1)
                    scatters(i0 + 1, 1, t, g)
                return 0

            lax.fori_loop(0, (nloc + 1) // 2, pair_body, 0)
            plsc.subcore_barrier()

            quarter = 2 * c + g
            pltpu.sync_copy(
                acc.at[pl.ds(base, ROWS_PER_TILE)],
                sums.at[t, quarter, pl.ds(base, ROWS_PER_TILE)])

            if g == 0:
                @pl.when(c == t % 2)
                def _():
                    pltpu.sync_copy(cnt.at[pl.ds(base, ROWS_PER_TILE)],
                                    cbuf)
                    pltpu.sync_copy(
                        cbuf, cnt_out.at[pl.ds(base, ROWS_PER_TILE)])

            plsc.subcore_barrier()

        for t, (h, ix, cnt_out) in enumerate(
                [(hs, ixs, cnt0), (hm, ixm, cnt1), (hf, ixf, cnt2)]):
            for g in range(2):
                one_group(t, g, h, ix, cnt_out)

    return agg(h_s, h_m, h_f, ix_s, ix_m, ix_f)


def _mlp(user_x, sums, cnts3, Wu, bu, Wfu1, bfu1, Wfu2, bfu2, Wc, bc):
    BLK = 1000

    def body(uxr, sr, cr, wur, bur, w1r, b1r, w2r, b2r, wcr, bcr,
             logitr, embr):
        uh = jnp.maximum(
            lax.dot_general(uxr[...], wur[...], (((1,), (1,)), ((), ())),
                            preferred_element_type=jnp.float32) + bur[...],
            0.0)
        means = []
        for t in range(3):
            m = jnp.concatenate([sr[t, 0], sr[t, 1], sr[t, 2], sr[t, 3]],
                                axis=1)
            cl = jnp.maximum(cr[t], 1.0)
            means.append(m / cl)
        fused = jnp.concatenate([uh] + means, axis=1)
        h = jnp.maximum(
            lax.dot_general(fused, w1r[...], (((1,), (1,)), ((), ())),
                            preferred_element_type=jnp.float32) + b1r[...],
            0.0)
        e = jnp.maximum(
            lax.dot_general(h, w2r[...], (((1,), (1,)), ((), ())),
                            preferred_element_type=jnp.float32) + b2r[...],
            0.0)
        embr[...] = e
        logitr[...] = jnp.sum(e * wcr[...], axis=1, keepdims=True) + bcr[...]

    logits2d, emb = pl.pallas_call(
        body,
        grid=(N_USERS // BLK,),
        in_specs=[
            pl.BlockSpec((BLK, D_IN), lambda i: (i, 0)),
            pl.BlockSpec((3, 4, BLK, QHALF), lambda i: (0, 0, i, 0)),
            pl.BlockSpec((3, BLK, 1), lambda i: (0, i, 0)),
            pl.BlockSpec((HID, D_IN), lambda i: (0, 0)),
            pl.BlockSpec((1, HID), lambda i: (0, 0)),
            pl.BlockSpec((HID, 4 * HID), lambda i: (0, 0)),
            pl.BlockSpec((1, HID), lambda i: (0, 0)),
            pl.BlockSpec((HID, HID), lambda i: (0, 0)),
            pl.BlockSpec((1, HID), lambda i: (0, 0)),
            pl.BlockSpec((1, HID), lambda i: (0, 0)),
            pl.BlockSpec((1, 1), lambda i: (0, 0)),
        ],
        out_specs=[
            pl.BlockSpec((BLK, 1), lambda i: (i, 0)),
            pl.BlockSpec((BLK, HID), lambda i: (i, 0)),
        ],
        out_shape=[
            jax.ShapeDtypeStruct((N_USERS, 1), jnp.float32),
            jax.ShapeDtypeStruct((N_USERS, HID), jnp.float32),
        ],
    )(user_x, sums, cnts3, Wu, bu.reshape(1, HID),
      Wfu1, bfu1.reshape(1, HID), Wfu2, bfu2.reshape(1, HID), Wc,
      bc.reshape(1, 1))
    return logits2d.reshape(N_USERS), emb


def kernel(user_x, session_x, message_x, feedback_x, session_to_user,
           message_to_user, feedback_to_user, Wu, bu, Ws, bs, Wm, bm, Wf, bf,
           Wfu1, bfu1, Wfu2, bfu2, Wc, bc):
    h_s = _proj(session_x, Ws, bs)
    h_m = _proj(message_x, Wm, bm)
    h_f = _proj(feedback_x, Wf, bf)
    ix_s = session_to_user.reshape(NCHUNK, KSUB, SUB)
    ix_m = message_to_user.reshape(NCHUNK, KSUB, SUB)
    ix_f = feedback_to_user.reshape(NCHUNK, KSUB, SUB)
    sums, c0, c1, c2 = _sc_aggregate(h_s, h_m, h_f, ix_s, ix_m, ix_f)
    cnts3 = jnp.stack([c0, c1, c2]).reshape(3, N_PAD, 1)
    return _mlp(user_x, sums, cnts3, Wu, bu, Wfu1, bfu1, Wfu2, bfu2, Wc, bc)


# speedup vs baseline: 4.6751x; 1.2701x over previous
import functools

import jax
import jax.numpy as jnp
from jax import lax
from jax.experimental import pallas as pl
from jax.experimental.pallas import tpu as pltpu
from jax.experimental.pallas import tpu_sc as plsc

N_USERS = 50000
N_PAD = 50048
N_EDGE = 500000
N_EDGE2 = N_EDGE // 2
D_IN = 128
HID = 64
HALF = 32
QHALF = 16
SUB = 125
KSUB = 8
CHUNK = SUB * KSUB
NCHUNK = N_EDGE // CHUNK
NCHUNK2 = NCHUNK // 2
NTILE = 16
ROWS_PER_TILE = N_PAD // NTILE


def _proj(x, W, b):
    N = x.shape[0]
    N2 = N // 2
    BLK = 2000
    NBLK2 = N2 // BLK

    def body(x1r, x2r, wr, br, outr):
        a1 = lax.dot_general(x1r[...], wr[...], (((1,), (1,)), ((), ())),
                             preferred_element_type=jnp.float32)
        outr[:, 0:HID] = jnp.maximum(a1 + br[...], 0.0)
        a2 = lax.dot_general(x2r[...], wr[...], (((1,), (1,)), ((), ())),
                             preferred_element_type=jnp.float32)
        outr[:, HID:2 * HID] = jnp.maximum(a2 + br[...], 0.0)

    return pl.pallas_call(
        body,
        grid=(NBLK2,),
        in_specs=[
            pl.BlockSpec((BLK, D_IN), lambda i: (i, 0)),
            pl.BlockSpec((BLK, D_IN), lambda i: (NBLK2 + i, 0)),
            pl.BlockSpec((HID, D_IN), lambda i: (0, 0)),
            pl.BlockSpec((1, HID), lambda i: (0, 0)),
        ],
        out_specs=pl.BlockSpec((BLK, 2 * HID), lambda i: (i, 0)),
        out_shape=jax.ShapeDtypeStruct((N2, 2 * HID), jnp.float32),
    )(x, x, W, b.reshape(1, HID))


def _sc_aggregate_one(h, ix, t):
    mesh = plsc.VectorSubcoreMesh(core_axis_name="c", subcore_axis_name="s")

    @functools.partial(
        pl.kernel,
        mesh=mesh,
        out_type=[
            jax.ShapeDtypeStruct((4, N_PAD, QHALF), jnp.float32),
            jax.ShapeDtypeStruct((N_PAD,), jnp.float32),
        ],
        scratch_types=[
            pltpu.VMEM((2, KSUB, SUB), jnp.int32),
            pltpu.VMEM((2, CHUNK, QHALF), jnp.float32),
            pltpu.VMEM((512, QHALF), jnp.float32),
            pltpu.VMEM((512,), jnp.float32),
            pltpu.VMEM((ROWS_PER_TILE,), jnp.float32),
            pltpu.VMEM((128,), jnp.float32),
            pltpu.VMEM_SHARED((N_PAD, QHALF), jnp.float32),
            pltpu.VMEM_SHARED((N_PAD,), jnp.float32),
            pltpu.SemaphoreType.DMA,
            pltpu.SemaphoreType.DMA,
        ],
        compiler_params=pltpu.CompilerParams(use_tc_tiling_on_sc=False),
    )
    def agg(h, ix, sums, cnt_out,
            idx_v, rows_v, zbuf, z1d, cbuf, ones_v, acc, cnt, sem0, sem1):
        c = lax.axis_index("c")
        s = lax.axis_index("s")
        zero16 = jnp.zeros((16,), jnp.float32)
        one16 = jnp.ones((16,), jnp.float32)

        def zrow(r, _):
            zbuf[r, pl.ds(0, 16)] = zero16
            return 0
        lax.fori_loop(0, 512, zrow, 0)

        def z1row(q, _):
            z1d[pl.ds(q * 16, 16)] = zero16
            return 0
        lax.fori_loop(0, 512 // 16, z1row, 0)

        for q in range(128 // 16):
            ones_v[pl.ds(q * 16, 16)] = one16

        base = s * ROWS_PER_TILE

        def one_group(g):
            for q in range(6):
                pltpu.sync_copy(zbuf, acc.at[pl.ds(base + q * 512, 512)])
            pltpu.sync_copy(zbuf.at[pl.ds(0, ROWS_PER_TILE - 6 * 512)],
                            acc.at[pl.ds(base + 6 * 512,
                                         ROWS_PER_TILE - 6 * 512)])

            if g == 0:
                @pl.when(c == t % 2)
                def _():
                    for q in range(6):
                        pltpu.sync_copy(
                            z1d, cnt.at[pl.ds(base + q * 512, 512)])
                    pltpu.sync_copy(
                        z1d.at[pl.ds(0, ROWS_PER_TILE - 6 * 512)],
                        cnt.at[pl.ds(base + 6 * 512,
                                     ROWS_PER_TILE - 6 * 512)])

            plsc.subcore_barrier()

            nloc = (NCHUNK - s + NTILE - 1) // NTILE
            col0 = c * HALF + g * QHALF
            sems = (sem0, sem1)

            def start_in(i, slot):
                j = s + i * NTILE
                pltpu.async_copy(ix.at[j], idx_v.at[slot], sems[slot])
                row = jnp.where(j < NCHUNK2, j, j - NCHUNK2) * CHUNK
                coff = jnp.where(j < NCHUNK2, col0, HID + col0)
                pltpu.async_copy(
                    h.at[pl.ds(row, CHUNK), pl.ds(coff, QHALF)],
                    rows_v.at[slot], sems[slot])

            def wait_in(slot):
                pltpu.make_async_copy(
                    ix.at[0], idx_v.at[slot], sems[slot]).wait()
                pltpu.make_async_copy(
                    h.at[pl.ds(0, CHUNK), pl.ds(0, QHALF)],
                    rows_v.at[slot], sems[slot]).wait()

            def scatters(i, slot, g):
                for k in range(KSUB):
                    pltpu.sync_copy(rows_v.at[slot, pl.ds(k * SUB, SUB)],
                                    acc.at[idx_v.at[slot, k]], add=True)
                if g == 0:
                    @pl.when(c == t % 2)
                    def _():
                        for k in range(KSUB):
                            pltpu.sync_copy(
                                ones_v.at[pl.ds(0, SUB)],
                                cnt.at[idx_v.at[slot, k]], add=True)

            @pl.when(nloc > 0)
            def _():
                start_in(0, 0)

            def pair_body(p, _):
                i0 = 2 * p

                @pl.when(i0 + 1 < nloc)
                def _():
                    start_in(i0 + 1, 1)
                wait_in(0)
                scatters(i0, 0, g)

                @pl.when(i0 + 2 < nloc)
                def _():
                    start_in(i0 + 2, 0)

                @pl.when(i0 + 1 < nloc)
                def _():
                    wait_in(1)
                    scatters(i0 + 1, 1, g)
                return 0

            lax.fori_loop(0, (nloc + 1) // 2, pair_body, 0)
            plsc.subcore_barrier()

            quarter = 2 * c + g
            pltpu.sync_copy(
                acc.at[pl.ds(base, ROWS_PER_TILE)],
                sums.at[quarter, pl.ds(base, ROWS_PER_TILE)])

            if g == 0:
                @pl.when(c == t % 2)
                def _():
                    pltpu.sync_copy(cnt.at[pl.ds(base, ROWS_PER_TILE)],
                                    cbuf)
                    pltpu.sync_copy(
                        cbuf, cnt_out.at[pl.ds(base, ROWS_PER_TILE)])

            plsc.subcore_barrier()

        one_group(0)
        one_group(1)

    return agg(h, ix)


def _mlp(user_x, sums3, cnts3, Wu, bu, Wfu1, bfu1, Wfu2, bfu2, Wc, bc):
    BLK = 1000

    def body(uxr, s0r, s1r, s2r, c0r, c1r, c2r, wur, bur, w1r, b1r, w2r,
             b2r, wcr, bcr, logitr, embr):
        uh = jnp.maximum(
            lax.dot_general(uxr[...], wur[...], (((1,), (1,)), ((), ())),
                            preferred_element_type=jnp.float32) + bur[...],
            0.0)
        means = []
        for sr, cr in ((s0r, c0r), (s1r, c1r), (s2r, c2r)):
            m = jnp.concatenate([sr[0], sr[1], sr[2], sr[3]],
                                axis=1)
            cl = jnp.maximum(cr[...], 1.0)
            means.append(m / cl)
        fused = jnp.concatenate([uh] + means, axis=1)
        h = jnp.maximum(
            lax.dot_general(fused, w1r[...], (((1,), (1,)), ((), ())),
                            preferred_element_type=jnp.float32) + b1r[...],
            0.0)
        e = jnp.maximum(
            lax.dot_general(h, w2r[...], (((1,), (1,)), ((), ())),
                            preferred_element_type=jnp.float32) + b2r[...],
            0.0)
        embr[...] = e
        logitr[...] = jnp.sum(e * wcr[...], axis=1, keepdims=True) + bcr[...]

    logits2d, emb = pl.pallas_call(
        body,
        grid=(N_USERS // BLK,),
        in_specs=[
            pl.BlockSpec((BLK, D_IN), lambda i: (i, 0)),
            pl.BlockSpec((4, BLK, QHALF), lambda i: (0, i, 0)),
            pl.BlockSpec((4, BLK, QHALF), lambda i: (0, i, 0)),
            pl.BlockSpec((4, BLK, QHALF), lambda i: (0, i, 0)),
            pl.BlockSpec((BLK, 1), lambda i: (i, 0)),
            pl.BlockSpec((BLK, 1), lambda i: (i, 0)),
            pl.BlockSpec((BLK, 1), lambda i: (i, 0)),
            pl.BlockSpec((HID, D_IN), lambda i: (0, 0)),
            pl.BlockSpec((1, HID), lambda i: (0, 0)),
            pl.BlockSpec((HID, 4 * HID), lambda i: (0, 0)),
            pl.BlockSpec((1, HID), lambda i: (0, 0)),
            pl.BlockSpec((HID, HID), lambda i: (0, 0)),
            pl.BlockSpec((1, HID), lambda i: (0, 0)),
            pl.BlockSpec((1, HID), lambda i: (0, 0)),
            pl.BlockSpec((1, 1), lambda i: (0, 0)),
        ],
        out_specs=[
            pl.BlockSpec((BLK, 1), lambda i: (i, 0)),
            pl.BlockSpec((BLK, HID), lambda i: (i, 0)),
        ],
        out_shape=[
            jax.ShapeDtypeStruct((N_USERS, 1), jnp.float32),
            jax.ShapeDtypeStruct((N_USERS, HID), jnp.float32),
        ],
    )(user_x, *sums3, *cnts3, Wu, bu.reshape(1, HID),
      Wfu1, bfu1.reshape(1, HID), Wfu2, bfu2.reshape(1, HID), Wc,
      bc.reshape(1, 1))
    return logits2d.reshape(N_USERS), emb


def kernel(user_x, session_x, message_x, feedback_x, session_to_user,
           message_to_user, feedback_to_user, Wu, bu, Ws, bs, Wm, bm, Wf, bf,
           Wfu1, bfu1, Wfu2, bfu2, Wc, bc):
    h_s = _proj(session_x, Ws, bs)
    h_m = _proj(message_x, Wm, bm)
    h_f = _proj(feedback_x, Wf, bf)
    ix_s = session_to_user.reshape(NCHUNK, KSUB, SUB)
    ix_m = message_to_user.reshape(NCHUNK, KSUB, SUB)
    ix_f = feedback_to_user.reshape(NCHUNK, KSUB, SUB)
    s0, c0 = _sc_aggregate_one(h_s, ix_s, 0)
    s1, c1 = _sc_aggregate_one(h_m, ix_m, 1)
    s2, c2 = _sc_aggregate_one(h_f, ix_f, 2)
    sums3 = (s0, s1, s2)
    cnts3 = (c0.reshape(N_PAD, 1), c1.reshape(N_PAD, 1),
             c2.reshape(N_PAD, 1))
    return _mlp(user_x, sums3, cnts3, Wu, bu, Wfu1, bfu1, Wfu2, bfu2,
                Wc, bc)
